# baseline (speedup 1.0000x reference)
import functools

import jax
import jax.numpy as jnp
from jax import lax
from jax.experimental import pallas as pl
from jax.experimental.pallas import tpu as pltpu
from jax.experimental.pallas import tpu_sc as plsc

N = 100000
E = 3200000
DIN = 20
H = 64

NC = 2
NS = 16
LANE = 128
ROWS = E // LANE
G = 8
NGROUPS = ROWS // G

NP = 100352
RPT = NP // NS

FW = 8
NPLANE = 4
PROWS = NP * 16 // 128

BLK = 2048
GRID = NP // BLK
PBLK = BLK * 16 // 128

_mesh = plsc.VectorSubcoreMesh(
    core_axis_name="c", subcore_axis_name="s", num_cores=NC, num_subcores=NS)
_sc_params = pltpu.CompilerParams(use_tc_tiling_on_sc=False)
_sc_params_nl = pltpu.CompilerParams(
    use_tc_tiling_on_sc=False, needs_layout_passes=False)


@functools.partial(
    pl.kernel,
    out_type=jax.ShapeDtypeStruct((NC * NS, NP), jnp.float32),
    mesh=_mesh,
    compiler_params=_sc_params_nl,
    scratch_types=[
        pltpu.VMEM((2 * G, LANE), jnp.int32),
        pltpu.VMEM((NP,), jnp.float32),
        pltpu.SemaphoreType.DMA,
    ],
)
def _sc_degree(dst2, degp, didx, acc, isem):
    c = lax.axis_index("c")
    s = lax.axis_index("s")
    w = s * NC + c
    per = (NGROUPS + NC * NS - 1) // (NC * NS)
    lo = w * per
    ng = jnp.maximum(0, jnp.minimum(per, NGROUPS - lo))

    z16 = jnp.zeros((16,), jnp.float32)

    def zbody(i, carry):
        acc[pl.ds(i * 16, 16)] = z16
        return carry

    lax.fori_loop(0, NP // 16, zbody, 0)

    def fire_idx(g, b):
        pltpu.async_copy(dst2.at[pl.ds((lo + g) * G, G)],
                         didx.at[pl.ds(b * G, G)], isem)

    def wait_idx():
        pltpu.make_async_copy(dst2.at[pl.ds(0, G)],
                              didx.at[pl.ds(0, G)], isem).wait()

    @pl.when(ng > 0)
    def _():
        fire_idx(0, 0)

    def step(i, carry):
        for b in range(2):
            g = 2 * i + b

            @pl.when(g < ng)
            def _():
                @pl.when(g + 1 < ng)
                def _():
                    fire_idx(g + 1, 1 - b)
                wait_idx()
                for r in range(G):
                    for k in range(LANE // 16):
                        dv = didx[b * G + r, pl.ds(k * 16, 16)]
                        cnt, last = plsc.scan_count(dv)
                        plsc.addupdate_scatter(
                            acc, [dv], cnt.astype(jnp.float32), mask=last)
        return carry

    lax.fori_loop(0, (ng + 1) // 2, step, 0)
    pltpu.sync_copy(acc, degp.at[w])


@functools.partial(
    pl.kernel,
    out_type=jax.ShapeDtypeStruct((NC, NPLANE, NP, FW), jnp.float32),
    mesh=_mesh,
    compiler_params=_sc_params,
    scratch_types=[
        pltpu.VMEM((4 * G, LANE), jnp.int32),
        pltpu.VMEM((4 * G, LANE), jnp.int32),
        pltpu.VMEM((4 * G, LANE, FW), jnp.float32),
        pltpu.VMEM_SHARED((NP, FW), jnp.float32),
        pltpu.SemaphoreType.DMA,
        pltpu.SemaphoreType.DMA,
        pltpu.SemaphoreType.DMA,
    ],
)
def _sc_aggregate(p4, srcp2, dst2, zrows, agg, sidx, didx, rows, acc,
                  isem, gsem, ssem):
    c = lax.axis_index("c")
    s = lax.axis_index("s")
    per = (NGROUPS + NS - 1) // NS
    lo = s * per
    ng = jnp.maximum(0, jnp.minimum(per, NGROUPS - lo))
    sl = pl.ds(s * RPT, RPT)

    def fire_idx(g, b):
        pltpu.async_copy(srcp2.at[c].at[pl.ds((lo + g) * G, G)],
                         sidx.at[pl.ds(b * G, G)], isem)
        pltpu.async_copy(dst2.at[pl.ds((lo + g) * G, G)],
                         didx.at[pl.ds(b * G, G)], isem)

    def wait_idx():
        for _ in range(2):
            pltpu.make_async_copy(dst2.at[pl.ds(0, G)],
                                  sidx.at[pl.ds(0, G)], isem).wait()

    def wait_gathers(j):
        for _ in range(G):
            pltpu.make_async_copy(p4.at[j].at[sidx.at[0]],
                                  rows.at[0], gsem).wait()

    def wait_scatters():
        for _ in range(G):
            pltpu.make_async_copy(rows.at[0], acc.at[didx.at[0]], ssem).wait()

    for j in range(NPLANE):
        pltpu.sync_copy(zrows, acc.at[sl])
        plsc.subcore_barrier()

        @pl.when(ng > 0)
        def _():
            fire_idx(0, 0)

        def stage(g):
            @pl.when(jnp.logical_and(g >= 3, g <= ng))
            def _():
                wait_scatters()

            @pl.when(g + 1 < ng)
            def _():
                fire_idx(g + 1, (g + 1) % 4)

            @pl.when(g < ng)
            def _():
                wait_idx()
                b = (g % 4) * G
                for r in range(G):
                    pltpu.async_copy(p4.at[j].at[sidx.at[b + r]],
                                     rows.at[b + r], gsem)

            @pl.when(jnp.logical_and(g >= 1, g <= ng))
            def _():
                wait_gathers(j)
                pb = ((g - 1) % 4) * G
                for r in range(G):
                    pltpu.async_copy(rows.at[pb + r],
                                     acc.at[didx.at[pb + r]],
                                     ssem, add=True)

        def step(i, carry):
            for u in range(4):
                stage(4 * i + u)
            return carry

        lax.fori_loop(0, (ng + 4) // 4, step, 0)

        @pl.when(ng > 1)
        def _():
            wait_scatters()

        @pl.when(ng > 0)
        def _():
            wait_scatters()

        plsc.subcore_barrier()
        pltpu.sync_copy(acc.at[sl], agg.at[c].at[j].at[sl])
        plsc.subcore_barrier()

def _row_mask(i):
    glob = i * BLK + lax.broadcasted_iota(jnp.int32, (BLK, 1), 0)
    return glob < N


def _tc1_body(x_ref, d_ref, W1_ref, h1_ref, p4_ref, dinv_ref):
    deg = jnp.sum(d_ref[...], axis=0) + 1.0
    dinv = 1.0 / jnp.sqrt(deg)
    h = jnp.dot(x_ref[...], W1_ref[...], preferred_element_type=jnp.float32)
    p = h * dinv[:, None]
    h1_ref[...] = h
    dinv_ref[...] = dinv
    p4_ref[...] = p


def _tc1(x, dall, W1):
    return pl.pallas_call(
        _tc1_body,
        grid=(GRID,),
        in_specs=[
            pl.BlockSpec((BLK, DIN), lambda i: (i, 0)),
            pl.BlockSpec((NC * NS, BLK), lambda i: (0, i)),
            pl.BlockSpec((DIN, H), lambda i: (0, 0)),
        ],
        out_specs=[
            pl.BlockSpec((BLK, H), lambda i: (i, 0)),
            pl.BlockSpec((BLK, H), lambda i: (i, 0)),
            pl.BlockSpec((BLK,), lambda i: (i,)),
        ],
        out_shape=[
            jax.ShapeDtypeStruct((NP, H), jnp.float32),
            jax.ShapeDtypeStruct((NP, H), jnp.float32),
            jax.ShapeDtypeStruct((NP,), jnp.float32),
        ],
    )(x, dall, W1)


def _tc2_body(agg4_ref, h_ref, dinv_ref, b_ref, out_ref, stats_ref):
    i = pl.program_id(0)
    agg = agg4_ref[...]
    dinv = dinv_ref[...][:, None]
    o = agg * dinv + h_ref[...] * (dinv * dinv) + b_ref[...][None, :]
    out_ref[...] = o
    om = jnp.where(_row_mask(i), o, 0.0)
    st = jnp.stack([jnp.sum(om, axis=0), jnp.sum(om * om, axis=0)])

    @pl.when(i == 0)
    def _():
        stats_ref[...] = st

    @pl.when(i > 0)
    def _():
        stats_ref[...] = stats_ref[...] + st


def _tc2(agg4, h, dinv, b):
    return pl.pallas_call(
        _tc2_body,
        grid=(GRID,),
        in_specs=[
            pl.BlockSpec((BLK, H), lambda i: (i, 0)),
            pl.BlockSpec((BLK, H), lambda i: (i, 0)),
            pl.BlockSpec((BLK,), lambda i: (i,)),
            pl.BlockSpec((H,), lambda i: (0,)),
        ],
        out_specs=[
            pl.BlockSpec((BLK, H), lambda i: (i, 0)),
            pl.BlockSpec((2, H), lambda i: (0, 0)),
        ],
        out_shape=[
            jax.ShapeDtypeStruct((NP, H), jnp.float32),
            jax.ShapeDtypeStruct((2, H), jnp.float32),
        ],
    )(agg4, h, dinv, b)


def _tc3_body(o_ref, st_ref, g_ref, be_ref, W2_ref, dinv_ref, h2_ref, p4_ref):
    st = st_ref[...]
    m = st[0] / N
    v = st[1] / N - m * m
    scale = g_ref[...] / jnp.sqrt(v + 1e-5)
    r = jnp.maximum((o_ref[...] - m[None, :]) * scale[None, :]
                    + be_ref[...][None, :], 0.0)
    h2 = jnp.dot(r, W2_ref[...], preferred_element_type=jnp.float32)
    p2 = h2 * dinv_ref[...][:, None]
    h2_ref[...] = h2
    p4_ref[...] = p2


def _tc3(out1, stats1, g1, be1, W2, dinv):
    return pl.pallas_call(
        _tc3_body,
        grid=(GRID,),
        in_specs=[
            pl.BlockSpec((BLK, H), lambda i: (i, 0)),
            pl.BlockSpec((2, H), lambda i: (0, 0)),
            pl.BlockSpec((H,), lambda i: (0,)),
            pl.BlockSpec((H,), lambda i: (0,)),
            pl.BlockSpec((H, H), lambda i: (0, 0)),
            pl.BlockSpec((BLK,), lambda i: (i,)),
        ],
        out_specs=[
            pl.BlockSpec((BLK, H), lambda i: (i, 0)),
            pl.BlockSpec((BLK, H), lambda i: (i, 0)),
        ],
        out_shape=[
            jax.ShapeDtypeStruct((NP, H), jnp.float32),
            jax.ShapeDtypeStruct((NP, H), jnp.float32),
        ],
    )(out1, stats1, g1, be1, W2, dinv)


def _tc5_body(o_ref, st_ref, g2_ref, be2_ref, A1_ref, ab1_ref, A2_ref,
              ab2_ref, M1_ref, mb1_ref, M2_ref, mb2_ref, out_ref, acc_ref):
    i = pl.program_id(0)
    st = st_ref[...]
    m = st[0] / N
    v = st[1] / N - m * m
    scale = g2_ref[...] / jnp.sqrt(v + 1e-5)
    r = jnp.maximum((o_ref[...] - m[None, :]) * scale[None, :]
                    + be2_ref[...][None, :], 0.0)
    t = jnp.dot(jnp.tanh(
        jnp.dot(r, A1_ref[...], preferred_element_type=jnp.float32)
        + ab1_ref[...][None, :]), A2_ref[...],
        preferred_element_type=jnp.float32) + ab2_ref[...][None, :]
    wgt = jnp.where(_row_mask(i), jnp.exp(t), 0.0)
    snum = jnp.sum(r * wgt, axis=0)
    sden = jnp.sum(wgt)
    st2 = jnp.stack([snum, jnp.full((H,), sden, jnp.float32)])

    @pl.when(i == 0)
    def _():
        acc_ref[...] = st2

    @pl.when(i > 0)
    def _():
        acc_ref[...] = acc_ref[...] + st2

    @pl.when(i == GRID - 1)
    def _():
        a = acc_ref[...]
        gvec = a[0:1, :] / a[1:2, 0:1]
        z = jnp.dot(
            jnp.maximum(jnp.dot(gvec, M1_ref[...],
                                preferred_element_type=jnp.float32)
                        + mb1_ref[...][None, :], 0.0),
            M2_ref[...], preferred_element_type=jnp.float32) \
            + mb2_ref[...][None, :]
        out_ref[...] = z[0]


def _tc5(out2, stats2, g2, be2, A1, ab1, A2, ab2, M1, mb1, M2, mb2):
    vec = lambda n: pl.BlockSpec((n,), lambda i: (0,))
    return pl.pallas_call(
        _tc5_body,
        grid=(GRID,),
        in_specs=[
            pl.BlockSpec((BLK, H), lambda i: (i, 0)),
            pl.BlockSpec((2, H), lambda i: (0, 0)),
            vec(H), vec(H),
            pl.BlockSpec((H, H), lambda i: (0, 0)), vec(H),
            pl.BlockSpec((H, 1), lambda i: (0, 0)), vec(1),
            pl.BlockSpec((H, H), lambda i: (0, 0)), vec(H),
            pl.BlockSpec((H, 1), lambda i: (0, 0)), vec(1),
        ],
        out_specs=[
            pl.BlockSpec((1,), lambda i: (0,)),
            pl.BlockSpec((2, H), lambda i: (0, 0)),
        ],
        out_shape=[
            jax.ShapeDtypeStruct((1,), jnp.float32),
            jax.ShapeDtypeStruct((2, H), jnp.float32),
        ],
    )(out2, stats2, g2, be2, A1, ab1, A2, ab2, M1, mb1, M2, mb2)


def kernel(x, edge_index, W1, b1, W2, b2, g1, be1, g2, be2,
           A1, ab1, A2, ab2, M1, mb1, M2, mb2):
    src0 = edge_index[0]
    srcp2 = jnp.stack([2 * src0, 2 * src0 + 1]).reshape(NC, ROWS, LANE)
    dst2 = edge_index[1].reshape(ROWS, LANE)
    zrows = jnp.zeros((RPT, FW), jnp.float32)
    xp = jnp.pad(x, ((0, NP - N), (0, 0)))

    def conv_agg(p):
        pv = p.reshape(NP, NPLANE, NC, FW).transpose(1, 0, 2, 3)
        pv = pv.reshape(NPLANE, NC * NP, FW)
        a = _sc_aggregate(pv, srcp2, dst2, zrows)
        return a.transpose(2, 1, 0, 3).reshape(NP, H)

    degp = _sc_degree(dst2)
    h1, p128_1, dinv = _tc1(xp, degp, W1)
    out1, stats1 = _tc2(conv_agg(p128_1), h1, dinv, b1)
    h2, p128_2 = _tc3(out1, stats1, g1, be1, W2, dinv)
    out2, stats2 = _tc2(conv_agg(p128_2), h2, dinv, b2)
    out, _ = _tc5(out2, stats2, g2, be2, A1, ab1, A2, ab2, M1, mb1, M2, mb2)
    return out


# speedup vs baseline: 1.1492x; 1.1492x over previous
import functools

import jax
import jax.numpy as jnp
from jax import lax
from jax.experimental import pallas as pl
from jax.experimental.pallas import tpu as pltpu
from jax.experimental.pallas import tpu_sc as plsc

N = 100000
E = 3200000
DIN = 20
H = 64

NC = 2
NS = 16
LANE = 128
ROWS = E // LANE
G = 8
NGROUPS = ROWS // G

NP = 100352
RPT = NP // NS

FW = 8
NPLANE = 4
PROWS = NP * 16 // 128

BLK = 2048
GRID = NP // BLK
PBLK = BLK * 16 // 128

_mesh = plsc.VectorSubcoreMesh(
    core_axis_name="c", subcore_axis_name="s", num_cores=NC, num_subcores=NS)
_sc_params = pltpu.CompilerParams(use_tc_tiling_on_sc=False)
_sc_params_nl = pltpu.CompilerParams(
    use_tc_tiling_on_sc=False, needs_layout_passes=False)


@functools.partial(
    pl.kernel,
    out_type=jax.ShapeDtypeStruct((NC * NS, NP), jnp.float32),
    mesh=_mesh,
    compiler_params=_sc_params_nl,
    scratch_types=[
        pltpu.VMEM((2 * G, LANE), jnp.int32),
        pltpu.VMEM((NP,), jnp.float32),
        pltpu.SemaphoreType.DMA,
    ],
)
def _sc_degree(dst2, degp, didx, acc, isem):
    c = lax.axis_index("c")
    s = lax.axis_index("s")
    w = s * NC + c
    per = (NGROUPS + NC * NS - 1) // (NC * NS)
    lo = w * per
    ng = jnp.maximum(0, jnp.minimum(per, NGROUPS - lo))

    z16 = jnp.zeros((16,), jnp.float32)

    def zbody(i, carry):
        acc[pl.ds(i * 16, 16)] = z16
        return carry

    lax.fori_loop(0, NP // 16, zbody, 0)

    def fire_idx(g, b):
        pltpu.async_copy(dst2.at[pl.ds((lo + g) * G, G)],
                         didx.at[pl.ds(b * G, G)], isem)

    def wait_idx():
        pltpu.make_async_copy(dst2.at[pl.ds(0, G)],
                              didx.at[pl.ds(0, G)], isem).wait()

    @pl.when(ng > 0)
    def _():
        fire_idx(0, 0)

    def step(i, carry):
        for b in range(2):
            g = 2 * i + b

            @pl.when(g < ng)
            def _():
                @pl.when(g + 1 < ng)
                def _():
                    fire_idx(g + 1, 1 - b)
                wait_idx()
                for r in range(G):
                    for k in range(LANE // 16):
                        dv = didx[b * G + r, pl.ds(k * 16, 16)]
                        cnt, last = plsc.scan_count(dv)
                        plsc.addupdate_scatter(
                            acc, [dv], cnt.astype(jnp.float32), mask=last)
        return carry

    lax.fori_loop(0, (ng + 1) // 2, step, 0)
    pltpu.sync_copy(acc, degp.at[w])


@functools.partial(
    pl.kernel,
    out_type=jax.ShapeDtypeStruct((NC, NPLANE, NP, FW), jnp.float32),
    mesh=_mesh,
    compiler_params=_sc_params,
    scratch_types=[
        pltpu.VMEM((6 * G, LANE), jnp.int32),
        pltpu.VMEM((6 * G, LANE), jnp.int32),
        pltpu.VMEM((6 * G, LANE, FW), jnp.float32),
        pltpu.VMEM_SHARED((NP, FW), jnp.float32),
        pltpu.SemaphoreType.DMA,
        pltpu.SemaphoreType.DMA,
        pltpu.SemaphoreType.DMA,
    ],
)
def _sc_aggregate(p4, srcp2, dst2, zrows, agg, sidx, didx, rows, acc,
                  isem, gsem, ssem):
    c = lax.axis_index("c")
    s = lax.axis_index("s")
    per = (NGROUPS + NS - 1) // NS
    lo = s * per
    ng = jnp.maximum(0, jnp.minimum(per, NGROUPS - lo))
    sl = pl.ds(s * RPT, RPT)

    def fire_idx(g, b):
        pltpu.async_copy(srcp2.at[c].at[pl.ds((lo + g) * G, G)],
                         sidx.at[pl.ds(b * G, G)], isem)
        pltpu.async_copy(dst2.at[pl.ds((lo + g) * G, G)],
                         didx.at[pl.ds(b * G, G)], isem)

    def wait_idx():
        for _ in range(2):
            pltpu.make_async_copy(dst2.at[pl.ds(0, G)],
                                  sidx.at[pl.ds(0, G)], isem).wait()

    def wait_gathers(j):
        for _ in range(G):
            pltpu.make_async_copy(p4.at[j].at[sidx.at[0]],
                                  rows.at[0], gsem).wait()

    def wait_scatters():
        for _ in range(G):
            pltpu.make_async_copy(rows.at[0], acc.at[didx.at[0]], ssem).wait()

    for j in range(NPLANE):
        pltpu.sync_copy(zrows, acc.at[sl])
        plsc.subcore_barrier()

        @pl.when(ng > 0)
        def _():
            fire_idx(0, 0)

        def stage(g):
            @pl.when(jnp.logical_and(g >= 5, g - 5 <= ng - 1))
            def _():
                wait_scatters()

            @pl.when(g + 1 < ng)
            def _():
                fire_idx(g + 1, (g + 1) % 6)

            @pl.when(g < ng)
            def _():
                wait_idx()
                b = (g % 6) * G
                for r in range(G):
                    pltpu.async_copy(p4.at[j].at[sidx.at[b + r]],
                                     rows.at[b + r], gsem)

            @pl.when(jnp.logical_and(g >= 3, g - 3 <= ng - 1))
            def _():
                wait_gathers(j)
                pb = ((g - 3) % 6) * G
                for r in range(G):
                    pltpu.async_copy(rows.at[pb + r],
                                     acc.at[didx.at[pb + r]],
                                     ssem, add=True)

        def step(i, carry):
            for u in range(4):
                stage(4 * i + u)
            return carry

        lax.fori_loop(0, (ng + 8) // 4, step, 0)

        plsc.subcore_barrier()
        pltpu.sync_copy(acc.at[sl], agg.at[c].at[j].at[sl])
        plsc.subcore_barrier()

def _row_mask(i):
    glob = i * BLK + lax.broadcasted_iota(jnp.int32, (BLK, 1), 0)
    return glob < N


def _tc1_body(x_ref, d_ref, W1_ref, h1_ref, p4_ref, dinv_ref):
    deg = jnp.sum(d_ref[...], axis=0) + 1.0
    dinv = 1.0 / jnp.sqrt(deg)
    h = jnp.dot(x_ref[...], W1_ref[...], preferred_element_type=jnp.float32)
    p = h * dinv[:, None]
    h1_ref[...] = h
    dinv_ref[...] = dinv
    p4_ref[...] = p


def _tc1(x, dall, W1):
    return pl.pallas_call(
        _tc1_body,
        grid=(GRID,),
        in_specs=[
            pl.BlockSpec((BLK, DIN), lambda i: (i, 0)),
            pl.BlockSpec((NC * NS, BLK), lambda i: (0, i)),
            pl.BlockSpec((DIN, H), lambda i: (0, 0)),
        ],
        out_specs=[
            pl.BlockSpec((BLK, H), lambda i: (i, 0)),
            pl.BlockSpec((BLK, H), lambda i: (i, 0)),
            pl.BlockSpec((BLK,), lambda i: (i,)),
        ],
        out_shape=[
            jax.ShapeDtypeStruct((NP, H), jnp.float32),
            jax.ShapeDtypeStruct((NP, H), jnp.float32),
            jax.ShapeDtypeStruct((NP,), jnp.float32),
        ],
    )(x, dall, W1)


def _tc2_body(agg4_ref, h_ref, dinv_ref, b_ref, out_ref, stats_ref):
    i = pl.program_id(0)
    agg = agg4_ref[...]
    dinv = dinv_ref[...][:, None]
    o = agg * dinv + h_ref[...] * (dinv * dinv) + b_ref[...][None, :]
    out_ref[...] = o
    om = jnp.where(_row_mask(i), o, 0.0)
    st = jnp.stack([jnp.sum(om, axis=0), jnp.sum(om * om, axis=0)])

    @pl.when(i == 0)
    def _():
        stats_ref[...] = st

    @pl.when(i > 0)
    def _():
        stats_ref[...] = stats_ref[...] + st


def _tc2(agg4, h, dinv, b):
    return pl.pallas_call(
        _tc2_body,
        grid=(GRID,),
        in_specs=[
            pl.BlockSpec((BLK, H), lambda i: (i, 0)),
            pl.BlockSpec((BLK, H), lambda i: (i, 0)),
            pl.BlockSpec((BLK,), lambda i: (i,)),
            pl.BlockSpec((H,), lambda i: (0,)),
        ],
        out_specs=[
            pl.BlockSpec((BLK, H), lambda i: (i, 0)),
            pl.BlockSpec((2, H), lambda i: (0, 0)),
        ],
        out_shape=[
            jax.ShapeDtypeStruct((NP, H), jnp.float32),
            jax.ShapeDtypeStruct((2, H), jnp.float32),
        ],
    )(agg4, h, dinv, b)


def _tc3_body(o_ref, st_ref, g_ref, be_ref, W2_ref, dinv_ref, h2_ref, p4_ref):
    st = st_ref[...]
    m = st[0] / N
    v = st[1] / N - m * m
    scale = g_ref[...] / jnp.sqrt(v + 1e-5)
    r = jnp.maximum((o_ref[...] - m[None, :]) * scale[None, :]
                    + be_ref[...][None, :], 0.0)
    h2 = jnp.dot(r, W2_ref[...], preferred_element_type=jnp.float32)
    p2 = h2 * dinv_ref[...][:, None]
    h2_ref[...] = h2
    p4_ref[...] = p2


def _tc3(out1, stats1, g1, be1, W2, dinv):
    return pl.pallas_call(
        _tc3_body,
        grid=(GRID,),
        in_specs=[
            pl.BlockSpec((BLK, H), lambda i: (i, 0)),
            pl.BlockSpec((2, H), lambda i: (0, 0)),
            pl.BlockSpec((H,), lambda i: (0,)),
            pl.BlockSpec((H,), lambda i: (0,)),
            pl.BlockSpec((H, H), lambda i: (0, 0)),
            pl.BlockSpec((BLK,), lambda i: (i,)),
        ],
        out_specs=[
            pl.BlockSpec((BLK, H), lambda i: (i, 0)),
            pl.BlockSpec((BLK, H), lambda i: (i, 0)),
        ],
        out_shape=[
            jax.ShapeDtypeStruct((NP, H), jnp.float32),
            jax.ShapeDtypeStruct((NP, H), jnp.float32),
        ],
    )(out1, stats1, g1, be1, W2, dinv)


def _tc5_body(o_ref, st_ref, g2_ref, be2_ref, A1_ref, ab1_ref, A2_ref,
              ab2_ref, M1_ref, mb1_ref, M2_ref, mb2_ref, out_ref, acc_ref):
    i = pl.program_id(0)
    st = st_ref[...]
    m = st[0] / N
    v = st[1] / N - m * m
    scale = g2_ref[...] / jnp.sqrt(v + 1e-5)
    r = jnp.maximum((o_ref[...] - m[None, :]) * scale[None, :]
                    + be2_ref[...][None, :], 0.0)
    t = jnp.dot(jnp.tanh(
        jnp.dot(r, A1_ref[...], preferred_element_type=jnp.float32)
        + ab1_ref[...][None, :]), A2_ref[...],
        preferred_element_type=jnp.float32) + ab2_ref[...][None, :]
    wgt = jnp.where(_row_mask(i), jnp.exp(t), 0.0)
    snum = jnp.sum(r * wgt, axis=0)
    sden = jnp.sum(wgt)
    st2 = jnp.stack([snum, jnp.full((H,), sden, jnp.float32)])

    @pl.when(i == 0)
    def _():
        acc_ref[...] = st2

    @pl.when(i > 0)
    def _():
        acc_ref[...] = acc_ref[...] + st2

    @pl.when(i == GRID - 1)
    def _():
        a = acc_ref[...]
        gvec = a[0:1, :] / a[1:2, 0:1]
        z = jnp.dot(
            jnp.maximum(jnp.dot(gvec, M1_ref[...],
                                preferred_element_type=jnp.float32)
                        + mb1_ref[...][None, :], 0.0),
            M2_ref[...], preferred_element_type=jnp.float32) \
            + mb2_ref[...][None, :]
        out_ref[...] = z[0]


def _tc5(out2, stats2, g2, be2, A1, ab1, A2, ab2, M1, mb1, M2, mb2):
    vec = lambda n: pl.BlockSpec((n,), lambda i: (0,))
    return pl.pallas_call(
        _tc5_body,
        grid=(GRID,),
        in_specs=[
            pl.BlockSpec((BLK, H), lambda i: (i, 0)),
            pl.BlockSpec((2, H), lambda i: (0, 0)),
            vec(H), vec(H),
            pl.BlockSpec((H, H), lambda i: (0, 0)), vec(H),
            pl.BlockSpec((H, 1), lambda i: (0, 0)), vec(1),
            pl.BlockSpec((H, H), lambda i: (0, 0)), vec(H),
            pl.BlockSpec((H, 1), lambda i: (0, 0)), vec(1),
        ],
        out_specs=[
            pl.BlockSpec((1,), lambda i: (0,)),
            pl.BlockSpec((2, H), lambda i: (0, 0)),
        ],
        out_shape=[
            jax.ShapeDtypeStruct((1,), jnp.float32),
            jax.ShapeDtypeStruct((2, H), jnp.float32),
        ],
    )(out2, stats2, g2, be2, A1, ab1, A2, ab2, M1, mb1, M2, mb2)


def kernel(x, edge_index, W1, b1, W2, b2, g1, be1, g2, be2,
           A1, ab1, A2, ab2, M1, mb1, M2, mb2):
    src0 = edge_index[0]
    srcp2 = jnp.stack([2 * src0, 2 * src0 + 1]).reshape(NC, ROWS, LANE)
    dst2 = edge_index[1].reshape(ROWS, LANE)
    zrows = jnp.zeros((RPT, FW), jnp.float32)
    xp = jnp.pad(x, ((0, NP - N), (0, 0)))

    def conv_agg(p):
        pv = p.reshape(NP, NPLANE, NC, FW).transpose(1, 0, 2, 3)
        pv = pv.reshape(NPLANE, NC * NP, FW)
        a = _sc_aggregate(pv, srcp2, dst2, zrows)
        return a.transpose(2, 1, 0, 3).reshape(NP, H)

    degp = _sc_degree(dst2)
    h1, p128_1, dinv = _tc1(xp, degp, W1)
    out1, stats1 = _tc2(conv_agg(p128_1), h1, dinv, b1)
    h2, p128_2 = _tc3(out1, stats1, g1, be1, W2, dinv)
    out2, stats2 = _tc2(conv_agg(p128_2), h2, dinv, b2)
    out, _ = _tc5(out2, stats2, g2, be2, A1, ab1, A2, ab2, M1, mb1, M2, mb2)
    return out
